# baseline (speedup 1.0000x reference)
import functools

import jax
import jax.numpy as jnp
from jax import lax
from jax.experimental import pallas as pl
from jax.experimental.pallas import tpu as pltpu
from jax.experimental.pallas import tpu_sc as plsc

N = 10000
E = 320000
D = 128
G = 64
NC = 2
NS = 16
NW = NC * NS
CHUNK = 128
NCHUNK = 78
NH = 13
HC = NCHUNK // NH
TAIL0 = NW * NCHUNK * CHUNK
NP = 10112
RPT = NP // NS
STAGES = ((0, 128), (128, 128), (256, 128), (384, 128), (512, 120))


def _sc_segment_sum(h, srcv, dstv, zrows):
  mesh = plsc.VectorSubcoreMesh(core_axis_name="c", subcore_axis_name="s")

  @functools.partial(
      pl.kernel,
      out_type=jax.ShapeDtypeStruct((NC, NP, D), jnp.float32),
      mesh=mesh,
      scratch_types=[
          pltpu.VMEM((2, 2, HC * CHUNK), jnp.int32),
          pltpu.VMEM((256, D), jnp.float32),
          pltpu.VMEM_SHARED((NP, D), jnp.float32),
          pltpu.SemaphoreType.DMA,
          pltpu.SemaphoreType.DMA,
          pltpu.SemaphoreType.DMA,
      ],
  )
  def k(h_hbm, src_hbm, dst_hbm, z_hbm, out_hbm,
        sd_l, rows, acc, sem0, sem1, isem):
    c = lax.axis_index("c")
    s = lax.axis_index("s")
    g = c * NS + s
    rows0 = rows.at[pl.ds(0, CHUNK)]
    rows1 = rows.at[pl.ds(128, CHUNK)]
    base = g * (NCHUNK * CHUNK)
    piece = HC * CHUNK
    pltpu.sync_copy(src_hbm.at[pl.ds(base, piece)],
                    sd_l.at[0, 0])
    pltpu.sync_copy(dst_hbm.at[pl.ds(base, piece)],
                    sd_l.at[0, 1])
    pltpu.async_copy(h_hbm.at[sd_l.at[0, 0, pl.ds(0, CHUNK)]], rows0, sem0)
    stage = rows.at[pl.ds(128, 128)]
    pltpu.sync_copy(z_hbm, stage)
    for off, ln in STAGES:
      pltpu.async_copy(stage.at[pl.ds(0, ln)],
                       acc.at[pl.ds(s * RPT + off, ln)], isem)
    for off, ln in STAGES:
      pltpu.make_async_copy(stage.at[pl.ds(0, ln)],
                            acc.at[pl.ds(s * RPT + off, ln)], isem).wait()
    plsc.subcore_barrier()

    for half in range(NH):
      b = half % 2
      src_l = sd_l.at[b, 0]
      dst_l = sd_l.at[b, 1]
      if half + 1 < NH:
        nb = (half + 1) % 2
        noff = base + (half + 1) * piece
        pltpu.async_copy(src_hbm.at[pl.ds(noff, piece)],
                         sd_l.at[nb, 0], isem)
        pltpu.async_copy(dst_hbm.at[pl.ds(noff, piece)],
                         sd_l.at[nb, 1], isem)

      def body(j2, carry):
        j = 2 * j2
        pltpu.async_copy(h_hbm.at[src_l.at[pl.ds((j + 1) * CHUNK, CHUNK)]], rows1, sem1)
        pltpu.make_async_copy(h_hbm.at[src_l.at[pl.ds(j * CHUNK, CHUNK)]], rows0, sem0).wait()
        pltpu.sync_copy(rows0, acc.at[dst_l.at[pl.ds(j * CHUNK, CHUNK)]], add=True)

        @pl.when(j2 < HC // 2 - 1)
        def _():
          pltpu.async_copy(h_hbm.at[src_l.at[pl.ds((j + 2) * CHUNK, CHUNK)]], rows0, sem0)

        pltpu.make_async_copy(h_hbm.at[src_l.at[pl.ds((j + 1) * CHUNK, CHUNK)]], rows1, sem1).wait()
        pltpu.sync_copy(rows1, acc.at[dst_l.at[pl.ds((j + 1) * CHUNK, CHUNK)]], add=True)
        return carry

      lax.fori_loop(0, HC // 2, body, 0)

      if half + 1 < NH:
        nb = (half + 1) % 2
        noff = base + (half + 1) * piece
        pltpu.make_async_copy(src_hbm.at[pl.ds(noff, piece)],
                              sd_l.at[nb, 0], isem).wait()
        pltpu.make_async_copy(dst_hbm.at[pl.ds(noff, piece)],
                              sd_l.at[nb, 1], isem).wait()
        pltpu.async_copy(h_hbm.at[sd_l.at[nb, 0, pl.ds(0, CHUNK)]], rows0, sem0)

    @pl.when(s < 2)
    def _tail():
      toff = TAIL0 + (c * 2 + s) * CHUNK
      tsrc = sd_l.at[0, 0, pl.ds(0, CHUNK)]
      tdst = sd_l.at[0, 1, pl.ds(0, CHUNK)]
      pltpu.sync_copy(src_hbm.at[pl.ds(toff, CHUNK)], tsrc)
      pltpu.sync_copy(dst_hbm.at[pl.ds(toff, CHUNK)], tdst)
      pltpu.async_copy(h_hbm.at[tsrc], rows0, sem0)
      pltpu.make_async_copy(h_hbm.at[tsrc], rows0, sem0).wait()
      pltpu.sync_copy(rows0, acc.at[tdst], add=True)
    plsc.subcore_barrier()

    osems = (sem0, sem1)
    for i, (off, ln) in enumerate(STAGES):
      r0 = s * RPT + off
      buf = rows.at[pl.ds(128 * (i % 2), ln)]
      if i >= 2:
        po, pln = STAGES[i - 2]
        pltpu.make_async_copy(rows.at[pl.ds(128 * (i % 2), pln)],
                              out_hbm.at[c, pl.ds(s * RPT + po, pln)],
                              osems[i % 2]).wait()
      pltpu.sync_copy(acc.at[pl.ds(r0, ln)], buf)
      pltpu.async_copy(buf, out_hbm.at[c, pl.ds(r0, ln)], osems[i % 2])
    for i in (len(STAGES) - 2, len(STAGES) - 1):
      off, ln = STAGES[i]
      pltpu.make_async_copy(rows.at[pl.ds(128 * (i % 2), ln)],
                            out_hbm.at[c, pl.ds(s * RPT + off, ln)],
                            osems[i % 2]).wait()

  return k(h, srcv, dstv, zrows)


def _layer_body(h_ref, p_ref, w1_ref, b1_ref, w2_ref, b2_ref, g_ref, be_ref):
  z = h_ref[...] + p_ref[0, pl.ds(0, N), :] + p_ref[1, pl.ds(0, N), :]
  a = jnp.maximum(
      jnp.dot(z, w1_ref[...], preferred_element_type=jnp.float32) + b1_ref[...],
      0.0)
  u = jnp.dot(a, w2_ref[...], preferred_element_type=jnp.float32) + b2_ref[...]
  mu = jnp.mean(u, axis=0, keepdims=True)
  d = u - mu
  var = jnp.mean(d * d, axis=0, keepdims=True)
  return jnp.maximum(g_ref[...] * d * lax.rsqrt(var + 1e-5) + be_ref[...], 0.0)


def _tc_layer(h, parts, W1, b1, W2, b2, gamma, beta):
  def body(h_ref, p_ref, w1_ref, b1_ref, w2_ref, b2_ref, g_ref, be_ref,
           out_ref):
    out_ref[...] = _layer_body(h_ref, p_ref, w1_ref, b1_ref, w2_ref, b2_ref,
                               g_ref, be_ref)

  return pl.pallas_call(
      body, out_shape=jax.ShapeDtypeStruct((N, D), jnp.float32))(
          h, parts, W1, b1, W2, b2, gamma, beta)


def _tc_layer_pool(h, parts, W1, b1, W2, b2, gamma, beta, batch2d):
  def body(h_ref, p_ref, w1_ref, b1_ref, w2_ref, b2_ref, g_ref, be_ref,
           bt_ref, out_ref):
    hv = _layer_body(h_ref, p_ref, w1_ref, b1_ref, w2_ref, b2_ref, g_ref,
                     be_ref)
    gid = lax.broadcasted_iota(jnp.int32, (G, N), 0)
    mask = (gid == bt_ref[...]).astype(jnp.float32)
    sums = lax.dot_general(mask, hv, (((1,), (0,)), ((), ())),
                           preferred_element_type=jnp.float32)
    counts = jnp.sum(mask, axis=1, keepdims=True)
    out_ref[...] = sums / jnp.maximum(counts, 1.0)

  return pl.pallas_call(
      body, out_shape=jax.ShapeDtypeStruct((G, D), jnp.float32))(
          h, parts, W1, b1, W2, b2, gamma, beta, batch2d)


def kernel(x, edge_index, batch,
           W1_0, b1_0, W2_0, b2_0, gamma_0, beta_0,
           W1_1, b1_1, W2_1, b2_1, gamma_1, beta_1):
  srcv, dstv = edge_index[0], edge_index[1]
  zrows = jnp.zeros((128, D), jnp.float32)
  batch2d = batch.reshape(1, N)
  b1_0r, b2_0r = b1_0.reshape(1, D), b2_0.reshape(1, D)
  b1_1r, b2_1r = b1_1.reshape(1, D), b2_1.reshape(1, D)
  g0, be0 = gamma_0.reshape(1, D), beta_0.reshape(1, D)
  g1, be1 = gamma_1.reshape(1, D), beta_1.reshape(1, D)

  p0 = _sc_segment_sum(x, srcv, dstv, zrows)
  h1 = _tc_layer(x, p0, W1_0, b1_0r, W2_0, b2_0r, g0, be0)
  p1 = _sc_segment_sum(h1, srcv, dstv, zrows)
  return _tc_layer_pool(h1, p1, W1_1, b1_1r, W2_1, b2_1r, g1, be1, batch2d)


# speedup vs baseline: 1.1050x; 1.1050x over previous
import functools

import jax
import jax.numpy as jnp
from jax import lax
from jax.experimental import pallas as pl
from jax.experimental.pallas import tpu as pltpu
from jax.experimental.pallas import tpu_sc as plsc

N = 10000
E = 320000
D = 128
G = 64
NC = 2
NS = 16
NW = NC * NS
EPT = E // NW
CHUNK = 125
NCHUNK = EPT // CHUNK
NH = 5
HC = NCHUNK // NH
NP = 10112
RPT = NP // NS
STAGES = ((0, 128), (128, 128), (256, 128), (384, 128), (512, 120))


def _sc_segment_sum(h, sd3, zrows):
  mesh = plsc.VectorSubcoreMesh(core_axis_name="c", subcore_axis_name="s")

  @functools.partial(
      pl.kernel,
      out_type=jax.ShapeDtypeStruct((NC, NP, D), jnp.float32),
      mesh=mesh,
      scratch_types=[
          pltpu.VMEM((2, 2, HC, CHUNK), jnp.int32),
          pltpu.VMEM((256, D), jnp.float32),
          pltpu.VMEM_SHARED((NP, D), jnp.float32),
          pltpu.SemaphoreType.DMA,
          pltpu.SemaphoreType.DMA,
          pltpu.SemaphoreType.DMA,
      ],
  )
  def k(h_hbm, sd_hbm, z_hbm, out_hbm,
        sd_l, rows, acc, sem0, sem1, isem):
    c = lax.axis_index("c")
    s = lax.axis_index("s")
    g = c * NS + s
    rows0 = rows.at[pl.ds(0, CHUNK)]
    rows1 = rows.at[pl.ds(128, CHUNK)]
    pltpu.sync_copy(sd_hbm.at[0, g, 0], sd_l.at[0, 0])
    pltpu.sync_copy(sd_hbm.at[1, g, 0], sd_l.at[0, 1])
    pltpu.async_copy(h_hbm.at[sd_l.at[0, 0, 0]], rows0, sem0)
    stage = rows.at[pl.ds(128, 128)]
    pltpu.sync_copy(z_hbm, stage)
    for off, ln in STAGES:
      pltpu.async_copy(stage.at[pl.ds(0, ln)],
                       acc.at[pl.ds(s * RPT + off, ln)], isem)
    for off, ln in STAGES:
      pltpu.make_async_copy(stage.at[pl.ds(0, ln)],
                            acc.at[pl.ds(s * RPT + off, ln)], isem).wait()
    plsc.subcore_barrier()

    for half in range(NH):
      b = half % 2
      src_l = sd_l.at[b, 0]
      dst_l = sd_l.at[b, 1]
      if half + 1 < NH:
        nb = (half + 1) % 2
        pltpu.async_copy(sd_hbm.at[0, g, half + 1], sd_l.at[nb, 0], isem)
        pltpu.async_copy(sd_hbm.at[1, g, half + 1], sd_l.at[nb, 1], isem)

      def body(j2, carry):
        j = 2 * j2
        pltpu.async_copy(h_hbm.at[src_l.at[j + 1]], rows1, sem1)
        pltpu.make_async_copy(h_hbm.at[src_l.at[j]], rows0, sem0).wait()
        pltpu.sync_copy(rows0, acc.at[dst_l.at[j]], add=True)

        @pl.when(j2 < HC // 2 - 1)
        def _():
          pltpu.async_copy(h_hbm.at[src_l.at[j + 2]], rows0, sem0)

        pltpu.make_async_copy(h_hbm.at[src_l.at[j + 1]], rows1, sem1).wait()
        pltpu.sync_copy(rows1, acc.at[dst_l.at[j + 1]], add=True)
        return carry

      lax.fori_loop(0, HC // 2, body, 0)

      if half + 1 < NH:
        nb = (half + 1) % 2
        pltpu.make_async_copy(sd_hbm.at[0, g, half + 1], sd_l.at[nb, 0],
                              isem).wait()
        pltpu.make_async_copy(sd_hbm.at[1, g, half + 1], sd_l.at[nb, 1],
                              isem).wait()
        pltpu.async_copy(h_hbm.at[sd_l.at[nb, 0, 0]], rows0, sem0)
    plsc.subcore_barrier()

    osems = (sem0, sem1)
    for i, (off, ln) in enumerate(STAGES):
      r0 = s * RPT + off
      buf = rows.at[pl.ds(128 * (i % 2), ln)]
      if i >= 2:
        po, pln = STAGES[i - 2]
        pltpu.make_async_copy(rows.at[pl.ds(128 * (i % 2), pln)],
                              out_hbm.at[c, pl.ds(s * RPT + po, pln)],
                              osems[i % 2]).wait()
      pltpu.sync_copy(acc.at[pl.ds(r0, ln)], buf)
      pltpu.async_copy(buf, out_hbm.at[c, pl.ds(r0, ln)], osems[i % 2])
    for i in (len(STAGES) - 2, len(STAGES) - 1):
      off, ln = STAGES[i]
      pltpu.make_async_copy(rows.at[pl.ds(128 * (i % 2), ln)],
                            out_hbm.at[c, pl.ds(s * RPT + off, ln)],
                            osems[i % 2]).wait()

  return k(h, sd3, zrows)


def _layer_body(h_ref, p_ref, w1_ref, b1_ref, w2_ref, b2_ref, g_ref, be_ref):
  z = h_ref[...] + p_ref[0, pl.ds(0, N), :] + p_ref[1, pl.ds(0, N), :]
  a = jnp.maximum(
      jnp.dot(z, w1_ref[...], preferred_element_type=jnp.float32) + b1_ref[...],
      0.0)
  u = jnp.dot(a, w2_ref[...], preferred_element_type=jnp.float32) + b2_ref[...]
  mu = jnp.mean(u, axis=0, keepdims=True)
  d = u - mu
  var = jnp.mean(d * d, axis=0, keepdims=True)
  return jnp.maximum(g_ref[...] * d * lax.rsqrt(var + 1e-5) + be_ref[...], 0.0)


def _tc_layer(h, parts, W1, b1, W2, b2, gamma, beta):
  def body(h_ref, p_ref, w1_ref, b1_ref, w2_ref, b2_ref, g_ref, be_ref,
           out_ref):
    out_ref[...] = _layer_body(h_ref, p_ref, w1_ref, b1_ref, w2_ref, b2_ref,
                               g_ref, be_ref)

  return pl.pallas_call(
      body, out_shape=jax.ShapeDtypeStruct((N, D), jnp.float32))(
          h, parts, W1, b1, W2, b2, gamma, beta)


def _tc_layer_pool(h, parts, W1, b1, W2, b2, gamma, beta, batch2d):
  def body(h_ref, p_ref, w1_ref, b1_ref, w2_ref, b2_ref, g_ref, be_ref,
           bt_ref, out_ref):
    hv = _layer_body(h_ref, p_ref, w1_ref, b1_ref, w2_ref, b2_ref, g_ref,
                     be_ref)
    gid = lax.broadcasted_iota(jnp.int32, (G, N), 0)
    mask = (gid == bt_ref[...]).astype(jnp.float32)
    sums = lax.dot_general(mask, hv, (((1,), (0,)), ((), ())),
                           preferred_element_type=jnp.float32)
    counts = jnp.sum(mask, axis=1, keepdims=True)
    out_ref[...] = sums / jnp.maximum(counts, 1.0)

  return pl.pallas_call(
      body, out_shape=jax.ShapeDtypeStruct((G, D), jnp.float32))(
          h, parts, W1, b1, W2, b2, gamma, beta, batch2d)


def kernel(x, edge_index, batch,
           W1_0, b1_0, W2_0, b2_0, gamma_0, beta_0,
           W1_1, b1_1, W2_1, b2_1, gamma_1, beta_1):
  sd3 = edge_index.reshape(2, NW, NH, HC, CHUNK)
  zrows = jnp.zeros((128, D), jnp.float32)
  batch2d = batch.reshape(1, N)
  b1_0r, b2_0r = b1_0.reshape(1, D), b2_0.reshape(1, D)
  b1_1r, b2_1r = b1_1.reshape(1, D), b2_1.reshape(1, D)
  g0, be0 = gamma_0.reshape(1, D), beta_0.reshape(1, D)
  g1, be1 = gamma_1.reshape(1, D), beta_1.reshape(1, D)

  p0 = _sc_segment_sum(x, sd3, zrows)
  h1 = _tc_layer(x, p0, W1_0, b1_0r, W2_0, b2_0r, g0, be0)
  p1 = _sc_segment_sum(h1, sd3, zrows)
  return _tc_layer_pool(h1, p1, W1_1, b1_1r, W2_1, b2_1r, g1, be1, batch2d)
